# baseline (speedup 1.0000x reference)
import functools

import jax
import jax.numpy as jnp
from jax import lax
from jax.experimental import pallas as pl
from jax.experimental.pallas import tpu as pltpu
from jax.experimental.pallas import tpu_sc as plsc

NC = 2
NS = 16
NW = NC * NS
L = 16

D = 64
SEQ = 200
TOTAL_ROWS = 4096 * 200
ROWS_PER_W = TOTAL_ROWS // NW
CHUNK = 256
N_CHUNKS = ROWS_PER_W // CHUNK
GPC = CHUNK // 128
PAIRS = CHUNK // 2
NBUF = 3
POS_PAIRS = SEQ // 2


def _body(idx_hbm, tt_hbm, posp_hbm, out_hbm,
          pos_v, idx_v, rows_v, idx_sem, gat_sem, out_sem):
    c = lax.axis_index("c")
    s = lax.axis_index("s")
    wid = s * NC + c
    tok0 = wid * ROWS_PER_W

    pltpu.sync_copy(posp_hbm, pos_v)

    def start_idx(j, b):
        for q in range(GPC):
            pltpu.async_copy(idx_hbm.at[pl.ds(tok0 + j * CHUNK + q * 128, 128)],
                             idx_v.at[b * GPC + q, 0], idx_sem.at[b])

    def wait_idx(b):
        for q in range(GPC):
            pltpu.make_async_copy(idx_hbm.at[pl.ds(0, 128)],
                                  idx_v.at[b * GPC + q, 0],
                                  idx_sem.at[b]).wait()

    def start_gathers(b):
        for q in range(GPC):
            pltpu.async_copy(tt_hbm.at[idx_v.at[b * GPC + q, 0]],
                             rows_v.at[b, pl.ds(q * 128, 128)],
                             gat_sem.at[b])

    def wait_gathers(b):
        for q in range(GPC):
            pltpu.make_async_copy(tt_hbm.at[pl.ds(0, 128)],
                                  rows_v.at[b, pl.ds(q * 128, 128)],
                                  gat_sem.at[b]).wait()

    def start_scatter(j, b):
        obase = pl.multiple_of((tok0 + j * CHUNK) // 2, PAIRS)
        pltpu.async_copy(rows_v.at[b, pl.ds(0, PAIRS)],
                         out_hbm.at[pl.ds(obase, PAIRS)],
                         out_sem.at[b])

    def wait_scatter(b):
        pltpu.make_async_copy(rows_v.at[b, pl.ds(0, PAIRS)],
                              out_hbm.at[pl.ds(0, PAIRS)],
                              out_sem.at[b]).wait()

    def compact_add(i, b):
        pair0 = i * PAIRS

        def body(p, carry):
            pr = lax.rem(pair0 + p, POS_PAIRS)
            for g in range(8):
                src_r = 2 * p + (g // 4)
                src_c = (g % 4) * L
                v = rows_v[b, src_r, pl.ds(src_c, L)] \
                    + pos_v[pr, pl.ds(g * L, L)]
                rows_v[b, p, pl.ds(g * L, L)] = v
            return carry

        lax.fori_loop(0, PAIRS, body, 0, unroll=2)

    start_idx(0, 0)
    start_idx(1, 1)
    wait_idx(0)
    start_gathers(0)

    def step(i, carry):
        b = i % NBUF
        nb = (i + 1) % NBUF

        @pl.when(i + 1 < N_CHUNKS)
        def _():
            wait_idx(nb)

            @pl.when(i + 2 < N_CHUNKS)
            def _():
                start_idx(i + 2, (i + 2) % NBUF)

            @pl.when(i + 1 >= NBUF)
            def _():
                wait_scatter(nb)

            start_gathers(nb)

        wait_gathers(b)
        compact_add(i, b)
        start_scatter(i, b)
        return carry

    lax.fori_loop(0, N_CHUNKS, step, 0)

    for t in range(NBUF):
        wait_scatter((N_CHUNKS - NBUF + t) % NBUF)


@jax.jit
def _run(idx_flat, tt_pad, pos_pairs):
    mesh = plsc.VectorSubcoreMesh(core_axis_name="c", subcore_axis_name="s",
                                  num_cores=NC, num_subcores=NS)
    return pl.kernel(
        _body,
        out_type=jax.ShapeDtypeStruct((TOTAL_ROWS // 2, 128), jnp.float32),
        mesh=mesh,
        scratch_types=[
            pltpu.VMEM((POS_PAIRS, 128), jnp.float32),
            pltpu.VMEM((NBUF * GPC, 8, 128), jnp.int32),
            pltpu.VMEM((NBUF, CHUNK, 128), jnp.float32),
            pltpu.SemaphoreType.DMA((NBUF,)),
            pltpu.SemaphoreType.DMA((NBUF,)),
            pltpu.SemaphoreType.DMA((NBUF,)),
        ],
        compiler_params=pltpu.CompilerParams(use_tc_tiling_on_sc=True),
    )(idx_flat, tt_pad, pos_pairs)


def kernel(x, token_table, pos_table):
    b, seq = x.shape
    idx_flat = x.reshape(-1).astype(jnp.int32)
    tt_pad = jnp.pad(token_table, ((0, 0), (0, 128 - D)))
    pos_pairs = pos_table.reshape(POS_PAIRS, 128)
    out = _run(idx_flat, tt_pad, pos_pairs)
    return out.reshape(b, seq, D)
